# baseline (speedup 1.0000x reference)
import functools

import jax
import jax.numpy as jnp
from jax import lax
from jax.experimental import pallas as pl
from jax.experimental.pallas import tpu as pltpu
from jax.experimental.pallas import tpu_sc as plsc

N = 10000
D = 128
NPAD = 10112
ROWS_PER_TILE = NPAD // 16
CH = 128
NTILES = 32
EPT = 10240
EPAD = NTILES * EPT
NCHUNK = EPT // CH
DUMMY = N


def _copy_idx(src, off, dst):
    for k in range(CH // 16):
        dst[pl.ds(k * 16, 16)] = src[pl.ds(off + k * 16, 16)]


def _unpack_idx(src, off, gdst, sdst):
    for k in range(CH // 16):
        v = src[pl.ds(off + k * 16, 16)]
        gdst[pl.ds(k * 16, 16)] = v & 0xFFFF
        sdst[pl.ds(k * 16, 16)] = v >> 16


def _hop_body(pk_h, h_h, zeros_h, out_h,
              pkv, gidx0, gidx1, sidx0, sidx1, gbuf0, gbuf1, acc,
              sem0, sem1, ssem0, ssem1):
    cid = lax.axis_index("c")
    sid = lax.axis_index("s")
    tid = cid * 16 + sid
    base = tid * EPT
    r0 = sid * ROWS_PER_TILE

    pltpu.sync_copy(pk_h.at[pl.ds(base, EPT + CH)], pkv)
    pltpu.sync_copy(zeros_h.at[pl.ds(r0, ROWS_PER_TILE)],
                    acc.at[pl.ds(r0, ROWS_PER_TILE)])
    plsc.subcore_barrier()

    _unpack_idx(pkv, 0, gidx0, sidx0)
    pltpu.make_async_copy(h_h.at[gidx0], gbuf0, sem0).start()

    def body(g, carry):
        off1 = (2 * g + 1) * CH
        off2 = (2 * g + 2) * CH

        @pl.when(g > 0)
        def _():
            pltpu.make_async_copy(gbuf1, acc.at[sidx1], ssem1).wait()
        _unpack_idx(pkv, off1, gidx1, sidx1)
        pltpu.make_async_copy(h_h.at[gidx1], gbuf1, sem1).start()
        pltpu.make_async_copy(h_h.at[gidx0], gbuf0, sem0).wait()
        pltpu.make_async_copy(gbuf0, acc.at[sidx0], ssem0).start(add=True)
        pltpu.make_async_copy(h_h.at[gidx1], gbuf1, sem1).wait()
        pltpu.make_async_copy(gbuf1, acc.at[sidx1], ssem1).start(add=True)
        pltpu.make_async_copy(gbuf0, acc.at[sidx0], ssem0).wait()
        _unpack_idx(pkv, off2, gidx0, sidx0)
        pltpu.make_async_copy(h_h.at[gidx0], gbuf0, sem0).start()
        return carry

    lax.fori_loop(0, NCHUNK // 2, body, 0)
    pltpu.make_async_copy(h_h.at[gidx0], gbuf0, sem0).wait()
    pltpu.make_async_copy(gbuf1, acc.at[sidx1], ssem1).wait()

    plsc.subcore_barrier()
    pltpu.sync_copy(acc.at[pl.ds(r0, ROWS_PER_TILE)],
                    out_h.at[cid, pl.ds(r0, ROWS_PER_TILE)])


def _make_hop():
    mesh = plsc.VectorSubcoreMesh(core_axis_name="c", subcore_axis_name="s")
    return functools.partial(
        pl.kernel, _hop_body, mesh=mesh,
        out_type=jax.ShapeDtypeStruct((2, NPAD, D), jnp.float32),
        scratch_types=[
            pltpu.VMEM((EPT + CH,), jnp.int32),
            pltpu.VMEM((CH,), jnp.int32),
            pltpu.VMEM((CH,), jnp.int32),
            pltpu.VMEM((CH,), jnp.int32),
            pltpu.VMEM((CH,), jnp.int32),
            pltpu.VMEM((CH, D), jnp.float32),
            pltpu.VMEM((CH, D), jnp.float32),
            pltpu.VMEM_SHARED((NPAD, D), jnp.float32),
            pltpu.SemaphoreType.DMA,
            pltpu.SemaphoreType.DMA,
            pltpu.SemaphoreType.DMA,
            pltpu.SemaphoreType.DMA,
        ],
    )()


def _deg_body(col_h, ones_h, zeros_h, out_h, colv, obuf,
              sidx0, sidx1, sidx2, sidx3, dacc, dsem):
    cid = lax.axis_index("c")
    sid = lax.axis_index("s")
    tid = cid * 16 + sid
    base = tid * EPT
    r0 = sid * ROWS_PER_TILE

    pltpu.sync_copy(col_h.at[pl.ds(base, EPT)], colv)
    pltpu.sync_copy(ones_h, obuf)
    pltpu.sync_copy(zeros_h.at[pl.ds(r0, ROWS_PER_TILE)],
                    dacc.at[pl.ds(r0, ROWS_PER_TILE)])
    plsc.subcore_barrier()

    sidxs = (sidx0, sidx1, sidx2, sidx3)

    def body(g, carry):
        for k in range(4):
            _copy_idx(colv, (4 * g + k) * CH, sidxs[k])
            pltpu.make_async_copy(obuf, dacc.at[sidxs[k]], dsem).start(add=True)
        for k in range(4):
            pltpu.make_async_copy(obuf, dacc.at[sidxs[k]], dsem).wait()
        return carry

    lax.fori_loop(0, NCHUNK // 4, body, 0)

    plsc.subcore_barrier()
    pltpu.sync_copy(dacc.at[pl.ds(r0, ROWS_PER_TILE)],
                    out_h.at[cid, pl.ds(r0, ROWS_PER_TILE)])


def _make_deg():
    mesh = plsc.VectorSubcoreMesh(core_axis_name="c", subcore_axis_name="s")
    return functools.partial(
        pl.kernel, _deg_body, mesh=mesh,
        out_type=jax.ShapeDtypeStruct((2, NPAD, D), jnp.float32),
        scratch_types=[
            pltpu.VMEM((EPT,), jnp.int32),
            pltpu.VMEM((CH, D), jnp.float32),
            pltpu.VMEM((CH,), jnp.int32),
            pltpu.VMEM((CH,), jnp.int32),
            pltpu.VMEM((CH,), jnp.int32),
            pltpu.VMEM((CH,), jnp.int32),
            pltpu.VMEM_SHARED((NPAD, D), jnp.float32),
            pltpu.SemaphoreType.DMA,
        ],
    )()


_RB = 2528


def _dinv_block(dega, degb):
    deg = dega[:, 0:1] + degb[:, 0:1] + 1.0
    return lax.rsqrt(deg), deg


def _prep_body(dega_r, degb_r, x_r, y0_r):
    dinv, _ = _dinv_block(dega_r[...], degb_r[...])
    y0_r[...] = x_r[...] * dinv


def _tc_prep(dega, degb, x):
    return pl.pallas_call(
        _prep_body,
        grid=(NPAD // _RB,),
        in_specs=[
            pl.BlockSpec((_RB, D), lambda i: (i, 0)),
            pl.BlockSpec((_RB, D), lambda i: (i, 0)),
            pl.BlockSpec((_RB, D), lambda i: (i, 0)),
        ],
        out_specs=pl.BlockSpec((_RB, D), lambda i: (i, 0)),
        out_shape=jax.ShapeDtypeStruct((NPAD, D), jnp.float32),
    )(dega, degb, x)


def _mid_body(dega_r, degb_r, za_r, zb_r, y0_r, y1_r):
    _, deg = _dinv_block(dega_r[...], degb_r[...])
    y1_r[...] = (za_r[...] + zb_r[...] + y0_r[...]) / deg


def _tc_mid(dega, degb, za, zb, y0):
    return pl.pallas_call(
        _mid_body,
        grid=(NPAD // _RB,),
        in_specs=[
            pl.BlockSpec((_RB, D), lambda i: (i, 0)),
            pl.BlockSpec((_RB, D), lambda i: (i, 0)),
            pl.BlockSpec((_RB, D), lambda i: (i, 0)),
            pl.BlockSpec((_RB, D), lambda i: (i, 0)),
            pl.BlockSpec((_RB, D), lambda i: (i, 0)),
        ],
        out_specs=pl.BlockSpec((_RB, D), lambda i: (i, 0)),
        out_shape=jax.ShapeDtypeStruct((NPAD, D), jnp.float32),
    )(dega, degb, za, zb, y0)


def _final_body(dega_r, degb_r, za_r, zb_r, y1_r, w_r, b_r, out_r):
    dinv, _ = _dinv_block(dega_r[...], degb_r[...])
    h = (za_r[...] + zb_r[...] + y1_r[...]) * dinv
    out_r[...] = lax.dot_general(
        h, w_r[...], (((1,), (1,)), ((), ())),
        preferred_element_type=jnp.float32) + b_r[...]


def _tc_final(dega, degb, za, zb, y1, W, b2):
    return pl.pallas_call(
        _final_body,
        grid=(NPAD // _RB,),
        in_specs=[
            pl.BlockSpec((_RB, D), lambda i: (i, 0)),
            pl.BlockSpec((_RB, D), lambda i: (i, 0)),
            pl.BlockSpec((_RB, D), lambda i: (i, 0)),
            pl.BlockSpec((_RB, D), lambda i: (i, 0)),
            pl.BlockSpec((_RB, D), lambda i: (i, 0)),
            pl.BlockSpec((D, D), lambda i: (0, 0)),
            pl.BlockSpec((1, D), lambda i: (0, 0)),
        ],
        out_specs=pl.BlockSpec((_RB, D), lambda i: (i, 0)),
        out_shape=jax.ShapeDtypeStruct((NPAD, D), jnp.float32),
    )(dega, degb, za, zb, y1, W, b2)


@jax.jit
def _run(x, edge_index, W, b):
    row = edge_index[0]
    col = edge_index[1]
    e = row.shape[0]
    pad = EPAD - e + CH
    fill = N + jnp.arange(pad, dtype=jnp.int32) % (NPAD - N)
    col_p = jnp.concatenate([col, fill])
    row_p = jnp.concatenate([row, fill])
    pk_p = (row_p << 16) | col_p

    x_p = jnp.pad(x, ((0, NPAD - N), (0, 0)))
    zeros = jnp.zeros((NPAD, D), jnp.float32)
    ones = jnp.ones((CH, D), jnp.float32)

    degs = _make_deg()(col_p, ones, zeros)
    dega, degb = degs[0], degs[1]

    y0 = _tc_prep(dega, degb, x_p)
    z1 = _make_hop()(pk_p, y0, zeros)
    y1 = _tc_mid(dega, degb, z1[0], z1[1], y0)
    z2 = _make_hop()(pk_p, y1, zeros)
    out = _tc_final(dega, degb, z2[0], z2[1], y1, W, b.reshape(1, D))
    return out[:N]


def kernel(x, edge_index, W, b):
    return _run(x, edge_index, W, b)


# speedup vs baseline: 1.2061x; 1.2061x over previous
import functools

import jax
import jax.numpy as jnp
from jax import lax
from jax.experimental import pallas as pl
from jax.experimental.pallas import tpu as pltpu
from jax.experimental.pallas import tpu_sc as plsc

N = 10000
D = 128
NPAD = 10112
ROWS_PER_TILE = NPAD // 16
CH = 128
NTILES = 32
EPT = 10240
EPAD = NTILES * EPT
NCHUNK = EPT // CH
DUMMY = N


def _copy_idx(src, off, dst):
    for k in range(CH // 16):
        dst[pl.ds(k * 16, 16)] = src[pl.ds(off + k * 16, 16)]


def _unpack_idx(src, off, gdst, sdst):
    for k in range(CH // 16):
        v = src[pl.ds(off + k * 16, 16)]
        gdst[pl.ds(k * 16, 16)] = v & 0xFFFF
        sdst[pl.ds(k * 16, 16)] = v >> 16


def _hop_body(pk_h, h_h, zeros_h, out_h,
              pkv, gidx0, gidx1, sidx0, sidx1, gbuf0, gbuf1, acc,
              sem0, sem1, ssem0, ssem1):
    cid = lax.axis_index("c")
    sid = lax.axis_index("s")
    tid = cid * 16 + sid
    base = tid * EPT
    r0 = sid * ROWS_PER_TILE

    pltpu.sync_copy(pk_h.at[pl.ds(base, EPT + CH)], pkv)
    pltpu.sync_copy(zeros_h.at[pl.ds(r0, ROWS_PER_TILE)],
                    acc.at[pl.ds(r0, ROWS_PER_TILE)])
    plsc.subcore_barrier()

    _unpack_idx(pkv, 0, gidx0, sidx0)
    pltpu.make_async_copy(h_h.at[gidx0], gbuf0, sem0).start()

    def body(g, carry):
        off1 = (2 * g + 1) * CH
        off2 = (2 * g + 2) * CH
        _unpack_idx(pkv, off1, gidx1, sidx1)
        pltpu.make_async_copy(h_h.at[gidx1], gbuf1, sem1).start()
        pltpu.make_async_copy(h_h.at[gidx0], gbuf0, sem0).wait()
        pltpu.sync_copy(gbuf0, acc.at[sidx0], add=True)
        _unpack_idx(pkv, off2, gidx0, sidx0)
        pltpu.make_async_copy(h_h.at[gidx0], gbuf0, sem0).start()
        pltpu.make_async_copy(h_h.at[gidx1], gbuf1, sem1).wait()
        pltpu.sync_copy(gbuf1, acc.at[sidx1], add=True)
        return carry

    lax.fori_loop(0, NCHUNK // 2, body, 0)
    pltpu.make_async_copy(h_h.at[gidx0], gbuf0, sem0).wait()

    plsc.subcore_barrier()
    pltpu.sync_copy(acc.at[pl.ds(r0, ROWS_PER_TILE)],
                    out_h.at[cid, pl.ds(r0, ROWS_PER_TILE)])


def _make_hop():
    mesh = plsc.VectorSubcoreMesh(core_axis_name="c", subcore_axis_name="s")
    return functools.partial(
        pl.kernel, _hop_body, mesh=mesh,
        out_type=jax.ShapeDtypeStruct((2, NPAD, D), jnp.float32),
        scratch_types=[
            pltpu.VMEM((EPT + CH,), jnp.int32),
            pltpu.VMEM((CH,), jnp.int32),
            pltpu.VMEM((CH,), jnp.int32),
            pltpu.VMEM((CH,), jnp.int32),
            pltpu.VMEM((CH,), jnp.int32),
            pltpu.VMEM((CH, D), jnp.float32),
            pltpu.VMEM((CH, D), jnp.float32),
            pltpu.VMEM_SHARED((NPAD, D), jnp.float32),
            pltpu.SemaphoreType.DMA,
            pltpu.SemaphoreType.DMA,
            pltpu.SemaphoreType.DMA,
            pltpu.SemaphoreType.DMA,
        ],
    )()


def _deg_body(col_h, ones_h, zeros_h, out_h, colv, obuf,
              sidx0, sidx1, sidx2, sidx3, dacc, dsem):
    cid = lax.axis_index("c")
    sid = lax.axis_index("s")
    tid = cid * 16 + sid
    base = tid * EPT
    r0 = sid * ROWS_PER_TILE

    pltpu.sync_copy(col_h.at[pl.ds(base, EPT)], colv)
    pltpu.sync_copy(ones_h, obuf)
    pltpu.sync_copy(zeros_h.at[pl.ds(r0, ROWS_PER_TILE)],
                    dacc.at[pl.ds(r0, ROWS_PER_TILE)])
    plsc.subcore_barrier()

    sidxs = (sidx0, sidx1, sidx2, sidx3)

    def body(g, carry):
        for k in range(4):
            _copy_idx(colv, (4 * g + k) * CH, sidxs[k])
            pltpu.make_async_copy(obuf, dacc.at[sidxs[k]], dsem).start(add=True)
        for k in range(4):
            pltpu.make_async_copy(obuf, dacc.at[sidxs[k]], dsem).wait()
        return carry

    lax.fori_loop(0, NCHUNK // 4, body, 0)

    plsc.subcore_barrier()
    pltpu.sync_copy(dacc.at[pl.ds(r0, ROWS_PER_TILE)],
                    out_h.at[cid, pl.ds(r0, ROWS_PER_TILE)])


def _make_deg():
    mesh = plsc.VectorSubcoreMesh(core_axis_name="c", subcore_axis_name="s")
    return functools.partial(
        pl.kernel, _deg_body, mesh=mesh,
        out_type=jax.ShapeDtypeStruct((2, NPAD, D), jnp.float32),
        scratch_types=[
            pltpu.VMEM((EPT,), jnp.int32),
            pltpu.VMEM((CH, D), jnp.float32),
            pltpu.VMEM((CH,), jnp.int32),
            pltpu.VMEM((CH,), jnp.int32),
            pltpu.VMEM((CH,), jnp.int32),
            pltpu.VMEM((CH,), jnp.int32),
            pltpu.VMEM_SHARED((NPAD, D), jnp.float32),
            pltpu.SemaphoreType.DMA,
        ],
    )()


_RB = 2528


def _dinv_block(dega, degb):
    deg = dega[:, 0:1] + degb[:, 0:1] + 1.0
    return lax.rsqrt(deg), deg


def _prep_body(dega_r, degb_r, x_r, y0_r):
    dinv, _ = _dinv_block(dega_r[...], degb_r[...])
    y0_r[...] = x_r[...] * dinv


def _tc_prep(dega, degb, x):
    return pl.pallas_call(
        _prep_body,
        grid=(NPAD // _RB,),
        in_specs=[
            pl.BlockSpec((_RB, D), lambda i: (i, 0)),
            pl.BlockSpec((_RB, D), lambda i: (i, 0)),
            pl.BlockSpec((_RB, D), lambda i: (i, 0)),
        ],
        out_specs=pl.BlockSpec((_RB, D), lambda i: (i, 0)),
        out_shape=jax.ShapeDtypeStruct((NPAD, D), jnp.float32),
    )(dega, degb, x)


def _mid_body(dega_r, degb_r, za_r, zb_r, y0_r, y1_r):
    _, deg = _dinv_block(dega_r[...], degb_r[...])
    y1_r[...] = (za_r[...] + zb_r[...] + y0_r[...]) / deg


def _tc_mid(dega, degb, za, zb, y0):
    return pl.pallas_call(
        _mid_body,
        grid=(NPAD // _RB,),
        in_specs=[
            pl.BlockSpec((_RB, D), lambda i: (i, 0)),
            pl.BlockSpec((_RB, D), lambda i: (i, 0)),
            pl.BlockSpec((_RB, D), lambda i: (i, 0)),
            pl.BlockSpec((_RB, D), lambda i: (i, 0)),
            pl.BlockSpec((_RB, D), lambda i: (i, 0)),
        ],
        out_specs=pl.BlockSpec((_RB, D), lambda i: (i, 0)),
        out_shape=jax.ShapeDtypeStruct((NPAD, D), jnp.float32),
    )(dega, degb, za, zb, y0)


def _final_body(dega_r, degb_r, za_r, zb_r, y1_r, w_r, b_r, out_r):
    dinv, _ = _dinv_block(dega_r[...], degb_r[...])
    h = (za_r[...] + zb_r[...] + y1_r[...]) * dinv
    out_r[...] = lax.dot_general(
        h, w_r[...], (((1,), (1,)), ((), ())),
        preferred_element_type=jnp.float32) + b_r[...]


def _tc_final(dega, degb, za, zb, y1, W, b2):
    return pl.pallas_call(
        _final_body,
        grid=(NPAD // _RB,),
        in_specs=[
            pl.BlockSpec((_RB, D), lambda i: (i, 0)),
            pl.BlockSpec((_RB, D), lambda i: (i, 0)),
            pl.BlockSpec((_RB, D), lambda i: (i, 0)),
            pl.BlockSpec((_RB, D), lambda i: (i, 0)),
            pl.BlockSpec((_RB, D), lambda i: (i, 0)),
            pl.BlockSpec((D, D), lambda i: (0, 0)),
            pl.BlockSpec((1, D), lambda i: (0, 0)),
        ],
        out_specs=pl.BlockSpec((_RB, D), lambda i: (i, 0)),
        out_shape=jax.ShapeDtypeStruct((NPAD, D), jnp.float32),
    )(dega, degb, za, zb, y1, W, b2)


@jax.jit
def _run(x, edge_index, W, b):
    row = edge_index[0]
    col = edge_index[1]
    e = row.shape[0]
    pad = EPAD - e + CH
    fill = N + jnp.arange(pad, dtype=jnp.int32) % (NPAD - N)
    col_p = jnp.concatenate([col, fill])
    row_p = jnp.concatenate([row, fill])
    pk_p = (row_p << 16) | col_p

    x_p = jnp.pad(x, ((0, NPAD - N), (0, 0)))
    zeros = jnp.zeros((NPAD, D), jnp.float32)
    ones = jnp.ones((CH, D), jnp.float32)

    degs = _make_deg()(col_p, ones, zeros)
    dega, degb = degs[0], degs[1]

    y0 = _tc_prep(dega, degb, x_p)
    z1 = _make_hop()(pk_p, y0, zeros)
    y1 = _tc_mid(dega, degb, z1[0], z1[1], y0)
    z2 = _make_hop()(pk_p, y1, zeros)
    out = _tc_final(dega, degb, z2[0], z2[1], y1, W, b.reshape(1, D))
    return out[:N]


def kernel(x, edge_index, W, b):
    return _run(x, edge_index, W, b)


# speedup vs baseline: 1.2827x; 1.0635x over previous
import functools

import jax
import jax.numpy as jnp
from jax import lax
from jax.experimental import pallas as pl
from jax.experimental.pallas import tpu as pltpu
from jax.experimental.pallas import tpu_sc as plsc

N = 10000
D = 128
NPAD = 10112
ROWS_PER_TILE = NPAD // 16
CH = 128
NTILES = 32
EPT = 10240
EPAD = NTILES * EPT
NCHUNK = EPT // CH
DUMMY = N


def _copy_idx(src, off, dst):
    for k in range(CH // 16):
        dst[pl.ds(k * 16, 16)] = src[pl.ds(off + k * 16, 16)]


def _unpack_idx(src, off, gdst, sdst):
    for k in range(CH // 16):
        v = src[pl.ds(off + k * 16, 16)]
        gdst[pl.ds(k * 16, 16)] = v & 0xFFFF
        sdst[pl.ds(k * 16, 16)] = v >> 16


def _hop_body(pk_h, h_h, zeros_h, out_h,
              pkv, gidx0, gidx1, sidx0, sidx1, gbuf0, gbuf1, acc,
              sem0, sem1, ssem0, ssem1):
    cid = lax.axis_index("c")
    sid = lax.axis_index("s")
    tid = cid * 16 + sid
    base = tid * EPT
    r0 = sid * ROWS_PER_TILE

    pltpu.sync_copy(pk_h.at[pl.ds(base, EPT + CH)], pkv)
    pltpu.sync_copy(zeros_h.at[pl.ds(r0, ROWS_PER_TILE)],
                    acc.at[pl.ds(r0, ROWS_PER_TILE)])
    plsc.subcore_barrier()

    _unpack_idx(pkv, 0, gidx0, sidx0)
    pltpu.make_async_copy(h_h.at[gidx0], gbuf0, sem0).start()

    def body(g, carry):
        off1 = (2 * g + 1) * CH
        off2 = (2 * g + 2) * CH
        _unpack_idx(pkv, off1, gidx1, sidx1)
        pltpu.make_async_copy(h_h.at[gidx1], gbuf1, sem1).start()
        pltpu.make_async_copy(h_h.at[gidx0], gbuf0, sem0).wait()
        pltpu.sync_copy(gbuf0, acc.at[sidx0], add=True)
        _unpack_idx(pkv, off2, gidx0, sidx0)
        pltpu.make_async_copy(h_h.at[gidx0], gbuf0, sem0).start()
        pltpu.make_async_copy(h_h.at[gidx1], gbuf1, sem1).wait()
        pltpu.sync_copy(gbuf1, acc.at[sidx1], add=True)
        return carry

    lax.fori_loop(0, NCHUNK // 2, body, 0)
    pltpu.make_async_copy(h_h.at[gidx0], gbuf0, sem0).wait()

    plsc.subcore_barrier()
    pltpu.sync_copy(acc.at[pl.ds(r0, ROWS_PER_TILE)],
                    out_h.at[cid, pl.ds(r0, ROWS_PER_TILE)])


def _make_hop():
    mesh = plsc.VectorSubcoreMesh(core_axis_name="c", subcore_axis_name="s")
    return functools.partial(
        pl.kernel, _hop_body, mesh=mesh,
        out_type=jax.ShapeDtypeStruct((2, NPAD, D), jnp.float32),
        scratch_types=[
            pltpu.VMEM((EPT + CH,), jnp.int32),
            pltpu.VMEM((CH,), jnp.int32),
            pltpu.VMEM((CH,), jnp.int32),
            pltpu.VMEM((CH,), jnp.int32),
            pltpu.VMEM((CH,), jnp.int32),
            pltpu.VMEM((CH, D), jnp.float32),
            pltpu.VMEM((CH, D), jnp.float32),
            pltpu.VMEM_SHARED((NPAD, D), jnp.float32),
            pltpu.SemaphoreType.DMA,
            pltpu.SemaphoreType.DMA,
            pltpu.SemaphoreType.DMA,
            pltpu.SemaphoreType.DMA,
        ],
    )()


def _deg_body(col_h, ones_h, zeros_h, out_h, colv, obuf,
              sidx0, sidx1, sidx2, sidx3, dacc, dsem):
    cid = lax.axis_index("c")
    sid = lax.axis_index("s")
    tid = cid * 16 + sid
    base = tid * EPT
    r0 = sid * ROWS_PER_TILE

    pltpu.sync_copy(col_h.at[pl.ds(base, EPT)], colv)
    pltpu.sync_copy(ones_h, obuf)
    pltpu.sync_copy(zeros_h.at[pl.ds(r0, ROWS_PER_TILE)],
                    dacc.at[pl.ds(r0, ROWS_PER_TILE)])
    plsc.subcore_barrier()

    sidxs = (sidx0, sidx1, sidx2, sidx3)

    def body(g, carry):
        for k in range(4):
            _copy_idx(colv, (4 * g + k) * CH, sidxs[k])
            pltpu.make_async_copy(obuf, dacc.at[sidxs[k]], dsem).start(add=True)
        for k in range(4):
            pltpu.make_async_copy(obuf, dacc.at[sidxs[k]], dsem).wait()
        return carry

    lax.fori_loop(0, NCHUNK // 4, body, 0)

    plsc.subcore_barrier()
    pltpu.sync_copy(dacc.at[pl.ds(r0, ROWS_PER_TILE)],
                    out_h.at[cid, pl.ds(r0, ROWS_PER_TILE)])


def _make_deg():
    mesh = plsc.VectorSubcoreMesh(core_axis_name="c", subcore_axis_name="s")
    return functools.partial(
        pl.kernel, _deg_body, mesh=mesh,
        out_type=jax.ShapeDtypeStruct((2, NPAD, D), jnp.float32),
        scratch_types=[
            pltpu.VMEM((EPT,), jnp.int32),
            pltpu.VMEM((CH, D), jnp.float32),
            pltpu.VMEM((CH,), jnp.int32),
            pltpu.VMEM((CH,), jnp.int32),
            pltpu.VMEM((CH,), jnp.int32),
            pltpu.VMEM((CH,), jnp.int32),
            pltpu.VMEM_SHARED((NPAD, D), jnp.float32),
            pltpu.SemaphoreType.DMA,
        ],
    )()


_RB = 2528


def _dinv_block(degs_a, degs_b):
    deg = degs_a[...][0, :, 0:1] + degs_b[...][0, :, 0:1] + 1.0
    return lax.rsqrt(deg), deg


def _prep_body(dega_r, degb_r, x_r, y0_r):
    dinv, _ = _dinv_block(dega_r, degb_r)
    y0_r[...] = x_r[...] * dinv


def _tc_prep(dega, degb, x):
    return pl.pallas_call(
        _prep_body,
        grid=(NPAD // _RB,),
        in_specs=[
            pl.BlockSpec((1, _RB, D), lambda i: (0, i, 0)),
            pl.BlockSpec((1, _RB, D), lambda i: (1, i, 0)),
            pl.BlockSpec((_RB, D), lambda i: (i, 0)),
        ],
        out_specs=pl.BlockSpec((_RB, D), lambda i: (i, 0)),
        out_shape=jax.ShapeDtypeStruct((NPAD, D), jnp.float32),
    )(dega, degb, x)


def _mid_body(dega_r, degb_r, za_r, zb_r, y0_r, y1_r):
    _, deg = _dinv_block(dega_r, degb_r)
    y1_r[...] = (za_r[...][0] + zb_r[...][0] + y0_r[...]) / deg


def _tc_mid(dega, degb, za, zb, y0):
    return pl.pallas_call(
        _mid_body,
        grid=(NPAD // _RB,),
        in_specs=[
            pl.BlockSpec((1, _RB, D), lambda i: (0, i, 0)),
            pl.BlockSpec((1, _RB, D), lambda i: (1, i, 0)),
            pl.BlockSpec((1, _RB, D), lambda i: (0, i, 0)),
            pl.BlockSpec((1, _RB, D), lambda i: (1, i, 0)),
            pl.BlockSpec((_RB, D), lambda i: (i, 0)),
        ],
        out_specs=pl.BlockSpec((_RB, D), lambda i: (i, 0)),
        out_shape=jax.ShapeDtypeStruct((NPAD, D), jnp.float32),
    )(dega, degb, za, zb, y0)


def _final_body(dega_r, degb_r, za_r, zb_r, y1_r, w_r, b_r, out_r):
    dinv, _ = _dinv_block(dega_r, degb_r)
    h = (za_r[...][0] + zb_r[...][0] + y1_r[...]) * dinv
    out_r[...] = lax.dot_general(
        h, w_r[...], (((1,), (1,)), ((), ())),
        preferred_element_type=jnp.float32) + b_r[...]


def _tc_final(dega, degb, za, zb, y1, W, b2):
    return pl.pallas_call(
        _final_body,
        grid=(NPAD // _RB,),
        in_specs=[
            pl.BlockSpec((1, _RB, D), lambda i: (0, i, 0)),
            pl.BlockSpec((1, _RB, D), lambda i: (1, i, 0)),
            pl.BlockSpec((1, _RB, D), lambda i: (0, i, 0)),
            pl.BlockSpec((1, _RB, D), lambda i: (1, i, 0)),
            pl.BlockSpec((_RB, D), lambda i: (i, 0)),
            pl.BlockSpec((D, D), lambda i: (0, 0)),
            pl.BlockSpec((1, D), lambda i: (0, 0)),
        ],
        out_specs=pl.BlockSpec((_RB, D), lambda i: (i, 0)),
        out_shape=jax.ShapeDtypeStruct((NPAD, D), jnp.float32),
    )(dega, degb, za, zb, y1, W, b2)


@jax.jit
def _run(x, edge_index, W, b):
    row = edge_index[0]
    col = edge_index[1]
    e = row.shape[0]
    pad = EPAD - e + CH
    fill = N + jnp.arange(pad, dtype=jnp.int32) % (NPAD - N)
    col_p = jnp.concatenate([col, fill])
    row_p = jnp.concatenate([row, fill])
    pk_p = (row_p << 16) | col_p

    x_p = jnp.pad(x, ((0, NPAD - N), (0, 0)))
    zeros = jnp.zeros((NPAD, D), jnp.float32)
    ones = jnp.ones((CH, D), jnp.float32)

    degs = _make_deg()(col_p, ones, zeros)

    y0 = _tc_prep(degs, degs, x_p)
    z1 = _make_hop()(pk_p, y0, zeros)
    y1 = _tc_mid(degs, degs, z1, z1, y0)
    z2 = _make_hop()(pk_p, y1, zeros)
    out = _tc_final(degs, degs, z2, z2, y1, W, b.reshape(1, D))
    return out[:N]


def kernel(x, edge_index, W, b):
    return _run(x, edge_index, W, b)
